# baseline (speedup 1.0000x reference)
import jax
import jax.numpy as jnp
from jax import lax
from jax.experimental import pallas as pl
from jax.experimental.pallas import tpu as pltpu
from jax.experimental.pallas import tpu_sc as plsc

H = 32
W = 32
D = 256
B = 32
F = 2 * D
LANES = 16


def _pos_body(row_hbm, col_hbm, out_hbm, rowv, build_v, sem):
    c = lax.axis_index("c")
    s = lax.axis_index("s")
    wid = c * 16 + s

    pltpu.sync_copy(col_hbm, build_v.at[:, pl.ds(0, D)])

    pltpu.sync_copy(row_hbm.at[pl.ds(wid, 1)], rowv)

    vs = [rowv[0, pl.ds(j * LANES, LANES)] for j in range(D // LANES)]

    def st(i, carry):
        for j in range(D // LANES):
            build_v[i, pl.ds(D + j * LANES, LANES)] = vs[j]
        return carry

    lax.fori_loop(0, W, st, 0)

    copies = [
        pltpu.async_copy(build_v, out_hbm.at[b, pl.ds(wid * W, W)], sem)
        for b in range(B)
    ]
    for cp in copies:
        cp.wait()


_pos_kernel = pl.kernel(
    _pos_body,
    out_type=jax.ShapeDtypeStruct((B, H * W, F), jnp.float32),
    mesh=plsc.VectorSubcoreMesh(core_axis_name="c", subcore_axis_name="s"),
    scratch_types=[
        pltpu.VMEM((1, D), jnp.float32),
        pltpu.VMEM((W, F), jnp.float32),
        pltpu.SemaphoreType.DMA,
    ],
)


def kernel(x, row_w, col_w):
    del x
    return _pos_kernel(row_w, col_w)
